# baseline (speedup 1.0000x reference)
import jax
import jax.numpy as jnp
from jax.experimental import pallas as pl
from jax.experimental.pallas import tpu as pltpu

_N = 4194304
_ROWS = 8192
_COLS = 2048
_BR = 256
_CHUNK = 512
_OUTC = _CHUNK // 4


def _cut_kernel(cut_ref, case_ref, x_ref, o_ref, s_ref):
    i = pl.program_id(0)

    @pl.when(i == 0)
    def _init():
        r = jax.lax.broadcasted_iota(jnp.int32, (_CHUNK, _OUTC), 0)
        c = jax.lax.broadcasted_iota(jnp.int32, (_CHUNK, _OUTC), 1)
        s_ref[...] = (r == c * 4).astype(jnp.float32)

    c0 = cut_ref[0]
    c1 = cut_ref[1]
    cs = case_ref[0]
    one = jnp.int32(1)
    zero = jnp.int32(0)
    sel = s_ref[...]
    for q in range(_COLS // _CHUNK):
        t = x_ref[:, q * _CHUNK:(q + 1) * _CHUNK]
        xf = jax.lax.dot(t, sel, precision=jax.lax.Precision.HIGHEST,
                         preferred_element_type=jnp.float32)
        p0 = jnp.where(xf <= c0, one, zero)
        p1 = jnp.where(xf >= c0, one, zero)
        p2 = jnp.where(jnp.logical_and(xf >= c0, xf <= c1), one, zero)
        p3 = jnp.where(jnp.logical_or(xf <= c0, xf >= c1), one, zero)
        o_ref[:, q * _OUTC:(q + 1) * _OUTC] = (
            jnp.where(cs == 0.0, p0,
            jnp.where(cs == 1.0, p1,
            jnp.where(cs == 2.0, p2, p3))))


def kernel(inputs, cut, case):
    xflat = inputs.reshape(_ROWS, _COLS)
    case1 = jnp.asarray(case, jnp.float32).reshape(1)
    out = pl.pallas_call(
        _cut_kernel,
        grid=(_ROWS // _BR,),
        in_specs=[
            pl.BlockSpec(memory_space=pltpu.SMEM),
            pl.BlockSpec(memory_space=pltpu.SMEM),
            pl.BlockSpec((_BR, _COLS), lambda i: (i, 0)),
        ],
        out_specs=pl.BlockSpec((_BR, _COLS // 4), lambda i: (i, 0)),
        out_shape=jax.ShapeDtypeStruct((_ROWS, _COLS // 4), jnp.int32),
        scratch_shapes=[pltpu.VMEM((_CHUNK, _OUTC), jnp.float32)],
        compiler_params=pltpu.CompilerParams(
            dimension_semantics=("arbitrary",),
        ),
    )(cut, case1, xflat)
    return out.reshape(_N)
